# baseline (speedup 1.0000x reference)
import jax
import jax.numpy as jnp
from jax import lax
from jax.experimental import pallas as pl
from jax.experimental.pallas import tpu as pltpu

C1 = 384
C2 = 384
C = 192
E = 4
TOPK = 2
H = 56
W = 56
HP = H + 2
WP = W + 2
NP = HP * WP
NI = (H - 1) * WP + W
OFF0 = WP + 1
STARTS = tuple(i * WP + j for i in range(3) for j in range(3))


def _silu(v):
    return v * jax.nn.sigmoid(v)


def _fused_body(x_ref, W1_ref, b1_ref, Wr_ref, br_ref, Wm_ref, be_ref,
                W2_ref, b2_ref, out_ref, y1p_ref):
    xb = x_ref[0].astype(jnp.bfloat16)
    t = jnp.dot(W1_ref[...], xb, preferred_element_type=jnp.float32) + b1_ref[...]
    t = _silu(t)
    y0 = t[:C].astype(jnp.bfloat16)
    y1 = t[C:]
    y1b = y1.astype(jnp.bfloat16)
    y1p_ref[...] = jnp.zeros((C, NP), jnp.bfloat16)
    for h in range(H):
        y1p_ref[:, (h + 1) * WP + 1:(h + 1) * WP + 1 + W] = y1b[:, h * W:(h + 1) * W]
    pooled = jnp.sum(y1, axis=1, keepdims=True) * jnp.float32(1.0 / (H * W))
    logits = jnp.dot(Wr_ref[...], pooled, preferred_element_type=jnp.float32) + br_ref[...]
    m = jnp.max(logits)
    ex = jnp.exp(logits - m)
    p = ex / jnp.sum(ex)
    io = lax.broadcasted_iota(jnp.int32, (E, 1), 0)
    p1 = jnp.max(p)
    i1 = jnp.min(jnp.where(p >= p1, io, E))
    pm = jnp.where(io == i1, -1.0, p)
    p2 = jnp.max(pm)
    i2 = jnp.min(jnp.where(pm >= p2, io, E))
    s = p1 + p2
    w1 = p1 / s
    w2 = p2 / s
    acc = None
    for sidx in range(9):
        wcat = jnp.concatenate([Wm_ref[i1, sidx], Wm_ref[i2, sidx]], axis=0)
        xs = y1p_ref[:, STARTS[sidx]:STARTS[sidx] + NI]
        d = jnp.dot(wcat, xs, preferred_element_type=jnp.float32)
        acc = d if acc is None else acc + d
    e1 = _silu(acc[:C] + be_ref[i1])
    e2 = _silu(acc[C:] + be_ref[i2])
    eo = w1 * e1 + w2 * e2
    eod = jnp.concatenate([eo[:, h * WP:h * WP + W] for h in range(H)], axis=1)
    o = (jnp.dot(W2_ref[:, :C], y0, preferred_element_type=jnp.float32)
         + jnp.dot(W2_ref[:, C:2 * C], y1b, preferred_element_type=jnp.float32)
         + jnp.dot(W2_ref[:, 2 * C:], eod.astype(jnp.bfloat16),
                   preferred_element_type=jnp.float32)
         + b2_ref[...])
    out_ref[0] = _silu(o)


def kernel(x, W1, b1, Wr, br, We, be, W2, b2):
    B = x.shape[0]
    xf = x.reshape(B, C1, H * W)
    W1r = W1.reshape(2 * C, C1).astype(jnp.bfloat16)
    b1c = b1.reshape(2 * C, 1)
    brc = br.reshape(E, 1)
    Wm = We.transpose(0, 3, 4, 1, 2).reshape(E, 9, C, C).astype(jnp.bfloat16)
    bec = be.reshape(E, C, 1)
    W2r = W2.reshape(C2, (2 + 1) * C).astype(jnp.bfloat16)
    b2c = b2.reshape(C2, 1)

    out_f = pl.pallas_call(
        _fused_body,
        grid=(B,),
        in_specs=[
            pl.BlockSpec((1, C1, H * W), lambda b: (b, 0, 0)),
            pl.BlockSpec((2 * C, C1), lambda b: (0, 0)),
            pl.BlockSpec((2 * C, 1), lambda b: (0, 0)),
            pl.BlockSpec((E, C), lambda b: (0, 0)),
            pl.BlockSpec((E, 1), lambda b: (0, 0)),
            pl.BlockSpec((E, 9, C, C), lambda b: (0, 0, 0, 0)),
            pl.BlockSpec((E, C, 1), lambda b: (0, 0, 0)),
            pl.BlockSpec((C2, 3 * C), lambda b: (0, 0)),
            pl.BlockSpec((C2, 1), lambda b: (0, 0)),
        ],
        out_specs=pl.BlockSpec((1, C2, H * W), lambda b: (b, 0, 0)),
        out_shape=jax.ShapeDtypeStruct((B, C2, H * W), jnp.float32),
        scratch_shapes=[pltpu.VMEM((C, NP), jnp.bfloat16)],
        compiler_params=pltpu.CompilerParams(
            dimension_semantics=("parallel",),
            vmem_limit_bytes=100 * 1024 * 1024),
    )(xf, W1r, b1c, Wr, brc, Wm, bec, W2r, b2c)

    return out_f.reshape(B, C2, H, W)


# speedup vs baseline: 1.0375x; 1.0375x over previous
import jax
import jax.numpy as jnp
from jax import lax
from jax.experimental import pallas as pl
from jax.experimental.pallas import tpu as pltpu

C1 = 384
C2 = 384
C = 192
E = 4
TOPK = 2
H = 56
W = 56
HP = H + 2
WP = W + 2
NP = HP * WP
NI = (H - 1) * WP + W
OFF0 = WP + 1
STARTS = tuple(i * WP + j for i in range(3) for j in range(3))


def _silu(v):
    return v * jax.nn.sigmoid(v)


def _fused_body(x_ref, W1_ref, b1_ref, Wr_ref, br_ref, Wm_ref, be_ref,
                W2_ref, b2_ref, out_ref, y1p_ref, xcat_ref, ycat_ref):
    xb = x_ref[0].astype(jnp.bfloat16)
    t = jnp.dot(W1_ref[...], xb, preferred_element_type=jnp.float32) + b1_ref[...]
    t = _silu(t)
    y1 = t[C:]
    ycat_ref[:C] = t[:C].astype(jnp.bfloat16)
    y1b = y1.astype(jnp.bfloat16)
    ycat_ref[C:2 * C] = y1b
    y1p_ref[...] = jnp.zeros((C, NP), jnp.bfloat16)
    for h in range(H):
        y1p_ref[:, (h + 1) * WP + 1:(h + 1) * WP + 1 + W] = y1b[:, h * W:(h + 1) * W]
    pooled = jnp.sum(y1, axis=1, keepdims=True) * jnp.float32(1.0 / (H * W))
    logits = jnp.dot(Wr_ref[...], pooled, preferred_element_type=jnp.float32) + br_ref[...]
    m = jnp.max(logits)
    ex = jnp.exp(logits - m)
    p = ex / jnp.sum(ex)
    io = lax.broadcasted_iota(jnp.int32, (E, 1), 0)
    p1 = jnp.max(p)
    i1 = jnp.min(jnp.where(p >= p1, io, E))
    pm = jnp.where(io == i1, -1.0, p)
    p2 = jnp.max(pm)
    i2 = jnp.min(jnp.where(pm >= p2, io, E))
    s = p1 + p2
    w1 = p1 / s
    w2 = p2 / s
    for sidx in range(9):
        xcat_ref[sidx * C:(sidx + 1) * C, :] = y1p_ref[:, STARTS[sidx]:STARTS[sidx] + NI]
    wcat = jnp.concatenate([Wm_ref[i1], Wm_ref[i2]], axis=0)
    acc = jnp.dot(wcat, xcat_ref[...], preferred_element_type=jnp.float32)
    e1 = _silu(acc[:C] + be_ref[i1])
    e2 = _silu(acc[C:] + be_ref[i2])
    eo = w1 * e1 + w2 * e2
    for h in range(H):
        ycat_ref[2 * C:, h * W:(h + 1) * W] = eo[:, h * WP:h * WP + W].astype(jnp.bfloat16)
    o = jnp.dot(W2_ref[...], ycat_ref[...], preferred_element_type=jnp.float32) + b2_ref[...]
    out_ref[0] = _silu(o)


def kernel(x, W1, b1, Wr, br, We, be, W2, b2):
    B = x.shape[0]
    xf = x.reshape(B, C1, H * W)
    W1r = W1.reshape(2 * C, C1).astype(jnp.bfloat16)
    b1c = b1.reshape(2 * C, 1)
    brc = br.reshape(E, 1)
    Wm2 = (We.transpose(0, 3, 4, 1, 2)
             .reshape(E, 9, C, C)
             .transpose(0, 2, 1, 3)
             .reshape(E, C, 9 * C)
             .astype(jnp.bfloat16))
    bec = be.reshape(E, C, 1)
    W2r = W2.reshape(C2, (2 + 1) * C).astype(jnp.bfloat16)
    b2c = b2.reshape(C2, 1)

    out_f = pl.pallas_call(
        _fused_body,
        grid=(B,),
        in_specs=[
            pl.BlockSpec((1, C1, H * W), lambda b: (b, 0, 0)),
            pl.BlockSpec((2 * C, C1), lambda b: (0, 0)),
            pl.BlockSpec((2 * C, 1), lambda b: (0, 0)),
            pl.BlockSpec((E, C), lambda b: (0, 0)),
            pl.BlockSpec((E, 1), lambda b: (0, 0)),
            pl.BlockSpec((E, C, 9 * C), lambda b: (0, 0, 0)),
            pl.BlockSpec((E, C, 1), lambda b: (0, 0, 0)),
            pl.BlockSpec((C2, 3 * C), lambda b: (0, 0)),
            pl.BlockSpec((C2, 1), lambda b: (0, 0)),
        ],
        out_specs=pl.BlockSpec((1, C2, H * W), lambda b: (b, 0, 0)),
        out_shape=jax.ShapeDtypeStruct((B, C2, H * W), jnp.float32),
        scratch_shapes=[
            pltpu.VMEM((C, NP), jnp.bfloat16),
            pltpu.VMEM((9 * C, NI), jnp.bfloat16),
            pltpu.VMEM((3 * C, H * W), jnp.bfloat16),
        ],
        compiler_params=pltpu.CompilerParams(
            dimension_semantics=("parallel",),
            vmem_limit_bytes=100 * 1024 * 1024),
    )(xf, W1r, b1c, Wr, brc, Wm2, bec, W2r, b2c)

    return out_f.reshape(B, C2, H, W)
